# Initial kernel scaffold; baseline (speedup 1.0000x reference)
import jax
import jax.numpy as jnp
from jax.experimental import pallas as pl


def kernel(x_low, x_high, attention):
    raise NotImplementedError("write your pallas kernel here")


# speedup vs baseline: 1.4554x; 1.4554x over previous
import jax
import jax.numpy as jnp
from jax.experimental import pallas as pl
from jax.experimental.pallas import tpu as pltpu

N_P = 16
PATCH = 96
HS = 128
WS = 128
HH = 1024
WH = 1024
CH = 3
NF = HS * WS


def _topk_kernel(att_ref, gum_ref, idx_ref, sa_ref, v_ref):
    att = att_ref[...]
    v_ref[...] = att + gum_ref[...]
    iota = jax.lax.broadcasted_iota(jnp.int32, att.shape, 1)
    for k in range(N_P):
        v = v_ref[...]
        m = jnp.max(v, axis=1, keepdims=True)
        idx = jnp.min(jnp.where(v == m, iota, NF), axis=1, keepdims=True)
        hit = iota == idx
        sa = jnp.sum(jnp.where(hit, att, 0.0), axis=1, keepdims=True)
        idx_ref[:, k : k + 1] = idx
        sa_ref[:, k : k + 1] = sa
        v_ref[...] = jnp.where(hit, -jnp.inf, v)


def _patch_kernel(sflat_ref, xh_ref, out_ref, scr_ref, sem):
    b = pl.program_id(0)
    p = pl.program_id(1)
    idx = sflat_ref[b * N_P + p]
    sx = idx // WS
    sy = idx - sx * WS
    r0 = sx * 8 - 44
    c0 = sy * 8 - 44
    r0a = pl.multiple_of(jnp.clip((r0 // 8) * 8, 0, HH - 104), 8)
    a0 = pl.multiple_of(jnp.clip((c0 // 128) * 128, 0, WH - 256), 128)

    copy = pltpu.make_async_copy(
        xh_ref.at[b, pl.ds(r0a, 104), pl.ds(pl.multiple_of(a0 * CH, 128), 256 * CH)],
        scr_ref,
        sem,
    )
    copy.start()
    k_i = jax.lax.broadcasted_iota(jnp.int32, (256 * CH, PATCH * CH), 0)
    c_i = jax.lax.broadcasted_iota(jnp.int32, (256 * CH, PATCH * CH), 1)
    tgt = CH * (c0 - a0 + jax.lax.rem(c_i, PATCH)) + c_i // PATCH
    sel = (k_i == tgt).astype(jnp.float32)
    i_i = jax.lax.broadcasted_iota(jnp.int32, (PATCH, 104), 0)
    r_i = jax.lax.broadcasted_iota(jnp.int32, (PATCH, 104), 1)
    rowsel = (r_i == i_i + (r0 - r0a)).astype(jnp.float32)
    copy.wait()

    slab = scr_ref[...]
    cmat = jnp.dot(slab, sel, preferred_element_type=jnp.float32)
    rmat = jnp.dot(rowsel, cmat, preferred_element_type=jnp.float32)
    for ch in range(CH):
        out_ref[0, 0, ch] = rmat[:, ch * PATCH : (ch + 1) * PATCH]


def kernel(x_low, x_high, attention):
    B = attention.shape[0]
    att2 = attention.reshape(B, NF)
    u = jax.random.uniform(jax.random.key(42), (B, NF), minval=1e-8, maxval=1.0)
    gum = -jnp.log(-jnp.log(u))

    sflat, sampled_att = pl.pallas_call(
        _topk_kernel,
        out_shape=[
            jax.ShapeDtypeStruct((B, N_P), jnp.int32),
            jax.ShapeDtypeStruct((B, N_P), jnp.float32),
        ],
        scratch_shapes=[pltpu.VMEM((B, NF), jnp.float32)],
    )(att2, gum)

    xh2 = x_high.reshape(B, HH, WH * CH)
    grid_spec = pltpu.PrefetchScalarGridSpec(
        num_scalar_prefetch=1,
        grid=(B, N_P),
        in_specs=[pl.BlockSpec(memory_space=pl.ANY)],
        out_specs=pl.BlockSpec(
            (1, 1, CH, PATCH, PATCH), lambda b, p, sref: (b, p, 0, 0, 0)
        ),
        scratch_shapes=[
            pltpu.VMEM((104, 256 * CH), jnp.float32),
            pltpu.SemaphoreType.DMA,
        ],
    )
    patches = pl.pallas_call(
        _patch_kernel,
        grid_spec=grid_spec,
        out_shape=jax.ShapeDtypeStruct((B, N_P, CH, PATCH, PATCH), jnp.float32),
    )(sflat.reshape(-1), xh2)
    return patches, sampled_att
